# baseline (speedup 1.0000x reference)
import functools

import jax
import jax.numpy as jnp
from jax import lax
from jax.experimental import pallas as pl
from jax.experimental.pallas import tpu as pltpu
from jax.experimental.pallas import tpu_sc as plsc


@functools.lru_cache(maxsize=None)
def _gather_fn(B, D, NC, NS, CH, NB, K):
    NW = NC * NS
    b_per_w = B // NW
    n_ch = b_per_w // CH
    n_grp = n_ch // NB
    mesh = plsc.VectorSubcoreMesh(core_axis_name="c", subcore_axis_name="s")

    @functools.partial(
        pl.kernel,
        mesh=mesh,
        out_type=jax.ShapeDtypeStruct((B, D), jnp.float32),
        scratch_types=[
            pltpu.VMEM((n_ch, CH), jnp.int32),
            pltpu.VMEM((NB, CH, D), jnp.float32),
        ]
        + [pltpu.SemaphoreType.DMA] * (2 * NB),
    )
    def k(table_hbm, idx_hbm, out_hbm, idx_v, rows_v, *sems):
        gsems, osems = sems[:NB], sems[NB:]
        wid = lax.axis_index("s") * NC + lax.axis_index("c")
        base = wid * b_per_w
        pltpu.sync_copy(idx_hbm.at[wid], idx_v)

        for b in range(K):
            pltpu.async_copy(table_hbm.at[idx_v.at[b]], rows_v.at[b], gsems[b])

        def body(jo, carry):
            for b in range(NB):
                j = jo * NB + b
                bp = (b + K) % NB

                @pl.when((j + K < n_ch) & (j + K >= NB))
                def _():
                    pltpu.make_async_copy(
                        rows_v.at[bp], out_hbm.at[pl.ds(base, CH)], osems[bp]
                    ).wait()

                @pl.when(j + K < n_ch)
                def _():
                    pltpu.async_copy(
                        table_hbm.at[idx_v.at[j + K]], rows_v.at[bp], gsems[bp]
                    )

                pltpu.make_async_copy(
                    table_hbm.at[idx_v.at[j]], rows_v.at[b], gsems[b]
                ).wait()
                pltpu.async_copy(
                    rows_v.at[b], out_hbm.at[pl.ds(base + j * CH, CH)], osems[b]
                )

            return carry

        lax.fori_loop(0, n_grp, body, 0)

        for b in range(NB):
            pltpu.make_async_copy(
                rows_v.at[b], out_hbm.at[pl.ds(base, CH)], osems[b]
            ).wait()

    return k


def kernel(idx, table):
    B0, S = idx.shape
    V, D = table.shape
    B = B0 * S
    info = plsc.get_sparse_core_info()
    NC, NS = info.num_cores, info.num_subcores
    NW = NC * NS
    CH = 128
    NB, K = 5, 2
    idx_flat = idx.reshape(B).astype(jnp.int32).reshape(NW, B // (NW * CH), CH)
    out = _gather_fn(B, D, NC, NS, CH, NB, K)(table, idx_flat)
    return out.reshape(B0, S, D)


# speedup vs baseline: 1.7934x; 1.7934x over previous
import functools

import jax
import jax.numpy as jnp
from jax import lax
from jax.experimental import pallas as pl
from jax.experimental.pallas import tpu as pltpu
from jax.experimental.pallas import tpu_sc as plsc


@functools.lru_cache(maxsize=None)
def _gather_fn(B0, S, D, NC, NS, CB, NB, K):
    NW = NC * NS
    b0_per_w = B0 // NW
    n_ch = b0_per_w // CB
    IC = CB * S
    n_grp = n_ch // NB
    mesh = plsc.VectorSubcoreMesh(core_axis_name="c", subcore_axis_name="s")

    @functools.partial(
        pl.kernel,
        mesh=mesh,
        out_type=jax.ShapeDtypeStruct((B0, S, D), jnp.float32),
        scratch_types=[
            pltpu.VMEM((n_ch, IC), jnp.int32),
            pltpu.VMEM((NB, IC, D), jnp.float32),
        ]
        + [pltpu.SemaphoreType.DMA] * (2 * NB),
    )
    def k(table_hbm, idx_hbm, out_hbm, idx_v, rows_v, *sems):
        gsems, osems = sems[:NB], sems[NB:]
        wid = lax.axis_index("s") * NC + lax.axis_index("c")
        b0base = wid * b0_per_w
        pltpu.sync_copy(idx_hbm.at[wid], idx_v)

        def wait_writes(b):
            for c in range(CB):
                pltpu.make_async_copy(
                    rows_v.at[b, pl.ds(c * S, S)], out_hbm.at[b0base], osems[b]
                ).wait()

        for b in range(K):
            pltpu.async_copy(table_hbm.at[idx_v.at[b]], rows_v.at[b], gsems[b])

        def body(jo, carry):
            for b in range(NB):
                j = jo * NB + b
                bp = (b + K) % NB

                @pl.when((j + K < n_ch) & (j + K >= NB))
                def _():
                    wait_writes(bp)

                @pl.when(j + K < n_ch)
                def _():
                    pltpu.async_copy(
                        table_hbm.at[idx_v.at[j + K]], rows_v.at[bp], gsems[bp]
                    )

                pltpu.make_async_copy(
                    table_hbm.at[idx_v.at[j]], rows_v.at[b], gsems[b]
                ).wait()
                for c in range(CB):
                    pltpu.async_copy(
                        rows_v.at[b, pl.ds(c * S, S)],
                        out_hbm.at[b0base + j * CB + c],
                        osems[b],
                    )

            return carry

        lax.fori_loop(0, n_grp, body, 0)

        for b in range(NB):
            wait_writes(b)

    return k


def kernel(idx, table):
    B0, S = idx.shape
    V, D = table.shape
    info = plsc.get_sparse_core_info()
    NC, NS = info.num_cores, info.num_subcores
    NW = NC * NS
    CB, NB, K = 2, 8, 4
    idx_w = idx.astype(jnp.int32).reshape(NW, (B0 // NW) // CB, CB * S)
    return _gather_fn(B0, S, D, NC, NS, CB, NB, K)(table, idx_w)
